# baseline (speedup 1.0000x reference)
import functools

import jax
import jax.numpy as jnp
from jax import lax
from jax.experimental import pallas as pl
from jax.experimental.pallas import tpu as pltpu
from jax.experimental.pallas import tpu_sc as plsc

_NC = 2
_NS = 16
_L = 16


def _tree_sum(vals):
    vals = list(vals)
    while len(vals) > 1:
        nxt = [vals[i] + vals[i + 1] for i in range(0, len(vals) - 1, 2)]
        if len(vals) % 2:
            nxt.append(vals[-1])
        vals = nxt
    return vals[0]


def _gather_segments(total):
    segs = []
    off = 0
    while off < total:
        cnt = min(128, total - off)
        segs.append((off, cnt))
        off += cnt
    return segs


def _splat(x):
    return jnp.broadcast_to(x, (_L,))


@jax.jit
def _relayout_tc(Ct):
    D, V = Ct.shape
    BE = 512

    def tk(x_ref, o_ref):
        xt = jnp.swapaxes(x_ref[...], 0, 1)
        o_ref[...] = jnp.concatenate(
            [xt, jnp.zeros((BE, 2 * D - D), jnp.float32)], axis=1)

    return pl.pallas_call(
        tk,
        grid=((V + BE - 1) // BE,),
        in_specs=[pl.BlockSpec((D, BE), lambda i: (0, i))],
        out_specs=pl.BlockSpec((BE, 2 * D), lambda i: (i, 0)),
        out_shape=jax.ShapeDtypeStruct((V, 2 * D), jnp.float32),
    )(Ct)


@jax.jit
def _relayout(Wt, Wtail):
    D, V = Wt.shape
    NW = _NC * _NS
    SR = V // 2
    NE = 512
    NFULL = V // NE
    TAIL = V - NFULL * NE
    ROUNDS = (NFULL + NW - 1) // NW

    mesh = plsc.VectorSubcoreMesh(
        core_axis_name="c", subcore_axis_name="s",
        num_cores=_NC, num_subcores=_NS,
    )

    @functools.partial(
        pl.kernel,
        out_type=jax.ShapeDtypeStruct((SR, 128), jnp.float32),
        mesh=mesh,
        compiler_params=pltpu.CompilerParams(use_tc_tiling_on_sc=True,
                                             needs_layout_passes=False),
        scratch_types=[
            pltpu.VMEM((D, NE), jnp.float32),
            pltpu.VMEM((D, NE), jnp.float32),
            pltpu.VMEM((NE // 2, 128), jnp.float32),
            pltpu.SemaphoreType.DMA,
            pltpu.SemaphoreType.DMA,
        ],
    )
    def k(wt_hbm, wtail_hbm, wc_hbm, in0, in1, out_v, sem0, sem1):
        wid = lax.axis_index("s") * _NC + lax.axis_index("c")
        ins = (in0, in1)
        sems = (sem0, sem1)
        srcs = (wt_hbm,)
        dsts = (wc_hbm,)

        def make_in(t, bid, slot, ne):
            return pltpu.make_async_copy(
                srcs[t].at[:, pl.ds(bid * NE, ne)],
                ins[slot].at[:, pl.ds(0, ne)], sems[slot])

        def transpose_block(t, bid, slot, ne):
            inb = ins[slot]
            lane = lax.iota(jnp.int32, _L)

            def cg_body(cg, _):
                c0 = cg * _L
                for di in range(D // _L):
                    r = [inb[di * _L + i, pl.ds(c0, _L)] for i in range(_L)]
                    for k in (8, 4, 2, 1):
                        m = (lane & k) == 0
                        for i in range(_L):
                            if i & k:
                                continue
                            a, b = r[i], r[i + k]
                            r[i] = jnp.where(
                                m, a, jnp.take(b, lane - k, mode="wrap"))
                            r[i + k] = jnp.where(
                                m, jnp.take(a, lane + k, mode="wrap"), b)
                    for l in range(_L):
                        out_v[cg * (_L // 2) + l // 2,
                              pl.ds((l % 2) * 64 + di * _L, _L)] = r[l]
                return 0

            lax.fori_loop(0, ne // _L, cg_body, 0)
            pltpu.sync_copy(
                out_v.at[pl.ds(0, ne // 2)],
                dsts[t].at[pl.ds(bid * (NE // 2), ne // 2)])

        for t in (0,):

            @pl.when(wid < NFULL)
            def _():
                make_in(t, wid, 0, NE).start()

            @pl.when(wid + NW < NFULL)
            def _():
                make_in(t, wid + NW, 1, NE).start()

            def body(i, _, t=t):
                for slot in (0, 1):
                    bid = wid + (i * 2 + slot) * NW

                    @pl.when(bid < NFULL)
                    def _():
                        make_in(t, bid, slot, NE).wait()
                        transpose_block(t, bid, slot, NE)

                    @pl.when(bid + 2 * NW < NFULL)
                    def _():
                        make_in(t, bid + 2 * NW, slot, NE).start()
                return 0

            lax.fori_loop(0, (ROUNDS + 1) // 2, body, 0)

        if TAIL:
            @pl.when(wid == 0)
            def _():
                pltpu.sync_copy(wtail_hbm,
                                wc_hbm.at[pl.ds(SR - TAIL // 2, TAIL // 2)])

    return k(Wt, Wtail)


@functools.partial(jax.jit, static_argnums=(4, 5, 6))
def _cbow_logits(ctx_idx, ci, W, C, B, CTX, NSCORE):
    D = W.shape[1]
    PC = C.shape[1]
    NW = _NC * _NS
    RW = B // NW
    NB = 8
    NCHUNK = RW // NB
    NG = D // _L
    OW = 2 * _L

    mesh = plsc.VectorSubcoreMesh(
        core_axis_name="c", subcore_axis_name="s",
        num_cores=_NC, num_subcores=_NS,
    )

    @functools.partial(
        pl.kernel,
        out_type=jax.ShapeDtypeStruct((B, OW), jnp.float32),
        mesh=mesh,
        compiler_params=pltpu.CompilerParams(use_tc_tiling_on_sc=False),
        scratch_types=[
            pltpu.VMEM((RW * CTX,), jnp.int32),
            pltpu.VMEM((RW * NSCORE,), jnp.int32),
            pltpu.VMEM((NB * CTX, D), jnp.float32),
            pltpu.VMEM((NB * CTX, D), jnp.float32),
            pltpu.VMEM((NB * NSCORE, PC), jnp.float32),
            pltpu.VMEM((NB * NSCORE, PC), jnp.float32),
            pltpu.VMEM((NB, OW), jnp.float32),
            pltpu.SemaphoreType.DMA,
            pltpu.SemaphoreType.DMA,
        ],
    )
    def k(ctx_hbm, ci_hbm, w_hbm, c_hbm, out_hbm,
          wi_all, ci_all, wrows0, wrows1, crows0, crows1, out_v, sem0, sem1):
        wid = lax.axis_index("s") * _NC + lax.axis_index("c")
        wrows = (wrows0, wrows1)
        crows = (crows0, crows1)
        sems = (sem0, sem1)

        pltpu.sync_copy(ctx_hbm.at[pl.ds(wid * RW * CTX, RW * CTX)], wi_all)
        pltpu.sync_copy(ci_hbm.at[pl.ds(wid * RW * NSCORE, RW * NSCORE)],
                        ci_all)

        def make_copies(ch, slot):
            cps = []
            for off, cnt in _gather_segments(NB * CTX):
                cps.append(pltpu.make_async_copy(
                    w_hbm.at[wi_all.at[pl.ds(ch * NB * CTX + off, cnt)]],
                    wrows[slot].at[pl.ds(off, cnt)],
                    sems[slot]))
            for off, cnt in _gather_segments(NB * NSCORE):
                cps.append(pltpu.make_async_copy(
                    c_hbm.at[ci_all.at[pl.ds(ch * NB * NSCORE + off, cnt)]],
                    crows[slot].at[pl.ds(off, cnt)],
                    sems[slot]))
            return cps

        def issue(ch, slot):
            for cp in make_copies(ch, slot):
                cp.start()

        def drain(ch, slot):
            for cp in make_copies(ch, slot):
                cp.wait()

        def compute(ch, slot):
            wr = wrows[slot]
            cr = crows[slot]

            def row_body(r, _):
                lane = lax.iota(jnp.int32, _L)
                wb = r * CTX
                cb = r * NSCORE
                ctx_e = []
                for g in range(NG):
                    gs = pl.ds(g * _L, _L)
                    ctx_e.append(
                        _tree_sum([wr[wb + j, gs] for j in range(CTX)])
                        * (1.0 / CTX))
                masked = [[] for _ in range(OW // _L)]
                for n in range(NSCORE):
                    p = _tree_sum([
                        cr[cb + n, pl.ds(g * _L, _L)] * ctx_e[g]
                        for g in range(NG)])
                    for sh in (8, 4, 2, 1):
                        p = p + jnp.take(p, lane ^ sh, mode="wrap")
                    masked[n // _L].append(
                        jnp.where(lane == (n % _L), p, 0.0))
                for v in range(OW // _L):
                    out_v[r, pl.ds(v * _L, _L)] = _tree_sum(masked[v])
                return 0

            lax.fori_loop(0, NB, row_body, 0)
            base = wid * RW + ch * NB
            pltpu.sync_copy(out_v, out_hbm.at[pl.ds(base, NB)])

        issue(0, 0)
        issue(1, 1)

        def pair_body(cp_i, _):
            for slot in (0, 1):
                ch = cp_i * 2 + slot
                drain(ch, slot)
                compute(ch, slot)

                @pl.when(ch + 2 < NCHUNK)
                def _():
                    issue(ch + 2, slot)
            return 0

        lax.fori_loop(0, NCHUNK // 2, pair_body, 0)

    return k(ctx_idx, ci, W, C)


def kernel(context, target, neg_samples, W, C):
    B, CTX = context.shape
    NEG = neg_samples.shape[1]
    ci = jnp.concatenate([target[:, None], neg_samples], axis=1).reshape(-1)
    V, D = W.shape
    tail_start = (V // 512) * 512
    Wc = _relayout(W.T, W[tail_start:].reshape(-1, 128))
    Cc = _relayout_tc(C.T)
    logits = _cbow_logits(context.reshape(-1), ci,
                          Wc.reshape(V, D), Cc, B, CTX, 1 + NEG)
    logits = logits[:, : 1 + NEG]
    labels = jnp.concatenate(
        [jnp.ones((B, 1), jnp.float32), jnp.zeros((B, NEG), jnp.float32)],
        axis=1,
    )
    return (logits, labels)


# speedup vs baseline: 1.7393x; 1.7393x over previous
import functools

import jax
import jax.numpy as jnp
from jax import lax
from jax.experimental import pallas as pl
from jax.experimental.pallas import tpu as pltpu
from jax.experimental.pallas import tpu_sc as plsc

_NC = 2
_NS = 16
_L = 16


def _tree_sum(vals):
    vals = list(vals)
    while len(vals) > 1:
        nxt = [vals[i] + vals[i + 1] for i in range(0, len(vals) - 1, 2)]
        if len(vals) % 2:
            nxt.append(vals[-1])
        vals = nxt
    return vals[0]


def _gather_segments(total):
    segs = []
    off = 0
    while off < total:
        cnt = min(128, total - off)
        segs.append((off, cnt))
        off += cnt
    return segs


def _splat(x):
    return jnp.broadcast_to(x, (_L,))


@jax.jit
def _relayout(Wt, Ct, Wtail, Ctail):
    D, V = Wt.shape
    NW = _NC * _NS
    SR = V // 2
    NE = 512
    NFULL = V // NE
    TAIL = V - NFULL * NE
    ROUNDS = (NFULL + NW - 1) // NW

    mesh = plsc.VectorSubcoreMesh(
        core_axis_name="c", subcore_axis_name="s",
        num_cores=_NC, num_subcores=_NS,
    )

    @functools.partial(
        pl.kernel,
        out_type=(jax.ShapeDtypeStruct((SR, 128), jnp.float32),
                  jax.ShapeDtypeStruct((SR, 128), jnp.float32)),
        mesh=mesh,
        compiler_params=pltpu.CompilerParams(use_tc_tiling_on_sc=True,
                                             needs_layout_passes=False),
        scratch_types=[
            pltpu.VMEM((D, NE), jnp.float32),
            pltpu.VMEM((D, NE), jnp.float32),
            pltpu.VMEM((NE // 2, 128), jnp.float32),
            pltpu.SemaphoreType.DMA,
            pltpu.SemaphoreType.DMA,
        ],
    )
    def k(wt_hbm, ct_hbm, wtail_hbm, ctail_hbm, wc_hbm, cc_hbm,
          in0, in1, out_v, sem0, sem1):
        wid = lax.axis_index("s") * _NC + lax.axis_index("c")
        ins = (in0, in1)
        sems = (sem0, sem1)
        srcs = (wt_hbm, ct_hbm)
        dsts = (wc_hbm, cc_hbm)

        def make_in(t, bid, slot, ne):
            return pltpu.make_async_copy(
                srcs[t].at[:, pl.ds(bid * NE, ne)],
                ins[slot].at[:, pl.ds(0, ne)], sems[slot])

        def transpose_block(t, bid, slot, ne):
            inb = ins[slot]
            lane = lax.iota(jnp.int32, _L)

            def cg_body(cg, _):
                c0 = cg * _L
                for di in range(D // _L):
                    r = [inb[di * _L + i, pl.ds(c0, _L)] for i in range(_L)]
                    for k in (8, 4, 2, 1):
                        m = (lane & k) == 0
                        for i in range(_L):
                            if i & k:
                                continue
                            a, b = r[i], r[i + k]
                            r[i] = jnp.where(
                                m, a, jnp.take(b, lane - k, mode="wrap"))
                            r[i + k] = jnp.where(
                                m, jnp.take(a, lane + k, mode="wrap"), b)
                    for l in range(_L):
                        out_v[cg * (_L // 2) + l // 2,
                              pl.ds((l % 2) * 64 + di * _L, _L)] = r[l]
                return 0

            lax.fori_loop(0, ne // _L, cg_body, 0)
            pltpu.sync_copy(
                out_v.at[pl.ds(0, ne // 2)],
                dsts[t].at[pl.ds(bid * (NE // 2), ne // 2)])

        for t in (0, 1):

            @pl.when(wid < NFULL)
            def _():
                make_in(t, wid, 0, NE).start()

            @pl.when(wid + NW < NFULL)
            def _():
                make_in(t, wid + NW, 1, NE).start()

            def body(i, _, t=t):
                for slot in (0, 1):
                    bid = wid + (i * 2 + slot) * NW

                    @pl.when(bid < NFULL)
                    def _():
                        make_in(t, bid, slot, NE).wait()
                        transpose_block(t, bid, slot, NE)

                    @pl.when(bid + 2 * NW < NFULL)
                    def _():
                        make_in(t, bid + 2 * NW, slot, NE).start()
                return 0

            lax.fori_loop(0, (ROUNDS + 1) // 2, body, 0)

        if TAIL:
            @pl.when(wid == 0)
            def _():
                pltpu.sync_copy(wtail_hbm,
                                wc_hbm.at[pl.ds(SR - TAIL // 2, TAIL // 2)])
                pltpu.sync_copy(ctail_hbm,
                                cc_hbm.at[pl.ds(SR - TAIL // 2, TAIL // 2)])

    return k(Wt, Ct, Wtail, Ctail)


@functools.partial(jax.jit, static_argnums=(4, 5, 6))
def _cbow_logits(ctx_idx, ci, W, C, B, CTX, NSCORE):
    D = W.shape[1]
    NW = _NC * _NS
    RW = B // NW
    NB = 16
    NCHUNK = RW // NB
    NG = D // _L
    OW = 2 * _L

    mesh = plsc.VectorSubcoreMesh(
        core_axis_name="c", subcore_axis_name="s",
        num_cores=_NC, num_subcores=_NS,
    )

    @functools.partial(
        pl.kernel,
        out_type=jax.ShapeDtypeStruct((B, OW), jnp.float32),
        mesh=mesh,
        compiler_params=pltpu.CompilerParams(use_tc_tiling_on_sc=False),
        scratch_types=[
            pltpu.VMEM((RW * CTX,), jnp.int32),
            pltpu.VMEM((RW * NSCORE,), jnp.int32),
            pltpu.VMEM((NB * CTX, D), jnp.float32),
            pltpu.VMEM((NB * CTX, D), jnp.float32),
            pltpu.VMEM((NB * NSCORE, D), jnp.float32),
            pltpu.VMEM((NB * NSCORE, D), jnp.float32),
            pltpu.VMEM((NB, OW), jnp.float32),
            pltpu.SemaphoreType.DMA,
            pltpu.SemaphoreType.DMA,
        ],
    )
    def k(ctx_hbm, ci_hbm, w_hbm, c_hbm, out_hbm,
          wi_all, ci_all, wrows0, wrows1, crows0, crows1, out_v, sem0, sem1):
        wid = lax.axis_index("s") * _NC + lax.axis_index("c")
        wrows = (wrows0, wrows1)
        crows = (crows0, crows1)
        sems = (sem0, sem1)

        pltpu.sync_copy(ctx_hbm.at[pl.ds(wid * RW * CTX, RW * CTX)], wi_all)
        pltpu.sync_copy(ci_hbm.at[pl.ds(wid * RW * NSCORE, RW * NSCORE)],
                        ci_all)

        def make_copies(ch, slot):
            cps = []
            for off, cnt in _gather_segments(NB * CTX):
                cps.append(pltpu.make_async_copy(
                    w_hbm.at[wi_all.at[pl.ds(ch * NB * CTX + off, cnt)]],
                    wrows[slot].at[pl.ds(off, cnt)],
                    sems[slot]))
            for off, cnt in _gather_segments(NB * NSCORE):
                cps.append(pltpu.make_async_copy(
                    c_hbm.at[ci_all.at[pl.ds(ch * NB * NSCORE + off, cnt)]],
                    crows[slot].at[pl.ds(off, cnt)],
                    sems[slot]))
            return cps

        def issue(ch, slot):
            for cp in make_copies(ch, slot):
                cp.start()

        def drain(ch, slot):
            for cp in make_copies(ch, slot):
                cp.wait()

        def compute(ch, slot):
            wr = wrows[slot]
            cr = crows[slot]

            def row_body(r, _):
                lane = lax.iota(jnp.int32, _L)
                wb = r * CTX
                cb = r * NSCORE
                ctx_e = []
                for g in range(NG):
                    gs = pl.ds(g * _L, _L)
                    ctx_e.append(
                        _tree_sum([wr[wb + j, gs] for j in range(CTX)])
                        * (1.0 / CTX))
                masked = [[] for _ in range(OW // _L)]
                for n in range(NSCORE):
                    p = _tree_sum([
                        cr[cb + n, pl.ds(g * _L, _L)] * ctx_e[g]
                        for g in range(NG)])
                    for sh in (8, 4, 2, 1):
                        p = p + jnp.take(p, lane ^ sh, mode="wrap")
                    masked[n // _L].append(
                        jnp.where(lane == (n % _L), p, 0.0))
                for v in range(OW // _L):
                    out_v[r, pl.ds(v * _L, _L)] = _tree_sum(masked[v])
                return 0

            lax.fori_loop(0, NB, row_body, 0)
            base = wid * RW + ch * NB
            pltpu.sync_copy(out_v, out_hbm.at[pl.ds(base, NB)])

        issue(0, 0)
        issue(1, 1)

        def pair_body(cp_i, _):
            for slot in (0, 1):
                ch = cp_i * 2 + slot
                drain(ch, slot)
                compute(ch, slot)

                @pl.when(ch + 2 < NCHUNK)
                def _():
                    issue(ch + 2, slot)
            return 0

        lax.fori_loop(0, NCHUNK // 2, pair_body, 0)

    return k(ctx_idx, ci, W, C)


def kernel(context, target, neg_samples, W, C):
    B, CTX = context.shape
    NEG = neg_samples.shape[1]
    ci = jnp.concatenate([target[:, None], neg_samples], axis=1).reshape(-1)
    V, D = W.shape
    tail_start = (V // 512) * 512
    Wc, Cc = _relayout(W.T, C.T,
                       W[tail_start:].reshape(-1, 128),
                       C[tail_start:].reshape(-1, 128))
    logits = _cbow_logits(context.reshape(-1), ci,
                          Wc.reshape(V, D), Cc.reshape(V, D), B, CTX, 1 + NEG)
    logits = logits[:, : 1 + NEG]
    labels = jnp.concatenate(
        [jnp.ones((B, 1), jnp.float32), jnp.zeros((B, NEG), jnp.float32)],
        axis=1,
    )
    return (logits, labels)
